# baseline (speedup 1.0000x reference)
import jax
import jax.numpy as jnp
from jax import lax
from jax.experimental import pallas as pl
from jax.experimental.pallas import tpu as pltpu
from jax.experimental.pallas import tpu_sc as plsc

NC = 2
NS = 16
NW = NC * NS
L = 16

B = 16384
E = 16
NB = 2
B_PER_W = B // NW
V = 1000000
TPB = V // 128
NT = TPB + 1
FLAT = NB * NT * 1024
CHUNK = 128
NCHUNK = B_PER_W // CHUNK
BATCH = 8


def _detile_body(uT_hbm, iT_hbm, xu_hbm, xi_hbm, uimg_hbm, iimg_hbm,
                 sem):
    wid = lax.axis_index("s") * NC + lax.axis_index("c")

    def step(n, carry):
        copies = []
        for t in range(BATCH):
            j = wid + 32 * (BATCH * n + t)
            jc = lax.min(j, TPB - 1)
            for b in range(NB):
                copies.append(pltpu.async_copy(
                    uT_hbm.at[b, :, pl.ds(jc * 128, 128)],
                    uimg_hbm.at[b * NT + jc], sem))
                copies.append(pltpu.async_copy(
                    iT_hbm.at[b, :, pl.ds(jc * 128, 128)],
                    iimg_hbm.at[b * NT + jc], sem))
        for cp in copies:
            cp.wait()
        return carry

    lax.fori_loop(0, (TPB + 32 * BATCH - 1) // (32 * BATCH), step, 0)

    @pl.when(wid == 0)
    def _():
        for b in range(NB):
            pltpu.async_copy(xu_hbm.at[b], uimg_hbm.at[b * NT + TPB],
                             sem).wait()
            pltpu.async_copy(xi_hbm.at[b], iimg_hbm.at[b * NT + TPB],
                             sem).wait()


def _gather_body(user_hbm, item_hbm, uflat_hbm, iflat_hbm, out_hbm,
                 uvidx, ividx, uaddr, iaddr, ugat, igat, outv, *sems):
    wid = lax.axis_index("s") * NC + lax.axis_index("c")
    base = wid * B_PER_W

    pltpu.sync_copy(user_hbm.at[wid], uvidx)
    pltpu.sync_copy(item_hbm.at[wid], ividx)

    econst = [(e // 8) * (NT * 1024) + (e % 8) * 128 for e in range(E)]

    def build(c, carry):
        for kk in range(CHUNK // L):
            sl = pl.ds(kk * L, L)
            for tab, idxr, addrr in ((0, uvidx, uaddr), (1, ividx, iaddr)):
                rvec = idxr[c, sl]
                t = lax.shift_right_logical(rvec, 7) * 1024 + (rvec & 127)
                for e in range(E):
                    addrr[c, e, sl] = t + econst[e]
        return carry

    lax.fori_loop(0, NCHUNK, build, 0)

    copies = []
    for c in range(NCHUNK):
        cs = []
        for e in range(E):
            cs.append(pltpu.async_copy(
                uflat_hbm.at[uaddr.at[c, e]], ugat.at[c, e], sems[c]))
            cs.append(pltpu.async_copy(
                iflat_hbm.at[iaddr.at[c, e]], igat.at[c, e], sems[c]))
        copies.append(cs)

    def group(c, g, carry):
        sl = pl.ds(g * L, L)
        acc = ugat[c, 0, sl] * igat[c, 0, sl]
        for e in range(1, E):
            acc = acc + ugat[c, e, sl] * igat[c, e, sl]
        outv[pl.ds(c * CHUNK + g * L, L)] = 1.0 / (1.0 + jnp.exp(-acc))
        return carry

    for c in range(NCHUNK):
        for cp in copies[c]:
            cp.wait()
        lax.fori_loop(0, CHUNK // L,
                      lambda g, k, _c=c: group(_c, g, k), 0)

    pltpu.sync_copy(outv, out_hbm.at[pl.ds(base, B_PER_W)])


def _mesh():
    return plsc.VectorSubcoreMesh(
        core_axis_name="c", subcore_axis_name="s",
        num_cores=NC, num_subcores=NS)


@jax.jit
def _gmf(user, item, uT3, iT3, xu, xi):
    uimg, iimg = pl.kernel(
        _detile_body,
        out_type=(jax.ShapeDtypeStruct((NB * NT, 8, 128), jnp.float32),
                  jax.ShapeDtypeStruct((NB * NT, 8, 128), jnp.float32)),
        mesh=_mesh(),
        scratch_types=[pltpu.SemaphoreType.DMA],
        compiler_params=pltpu.CompilerParams(use_tc_tiling_on_sc=True),
    )(uT3, iT3, xu, xi)

    out = pl.kernel(
        _gather_body,
        out_type=jax.ShapeDtypeStruct((B,), jnp.float32),
        mesh=_mesh(),
        scratch_types=[
            pltpu.VMEM((NCHUNK, CHUNK), jnp.int32),
            pltpu.VMEM((NCHUNK, CHUNK), jnp.int32),
            pltpu.VMEM((NCHUNK, E, CHUNK), jnp.int32),
            pltpu.VMEM((NCHUNK, E, CHUNK), jnp.int32),
            pltpu.VMEM((NCHUNK, E, CHUNK), jnp.float32),
            pltpu.VMEM((NCHUNK, E, CHUNK), jnp.float32),
            pltpu.VMEM((B_PER_W,), jnp.float32),
        ] + [pltpu.SemaphoreType.DMA] * NCHUNK,
        compiler_params=pltpu.CompilerParams(use_tc_tiling_on_sc=False),
    )(user, item, uimg.reshape(FLAT), iimg.reshape(FLAT))
    return out


def kernel(user, item, user_embedding, item_embedding):
    u = user.astype(jnp.int32).reshape(NW, NCHUNK, CHUNK)
    i = item.astype(jnp.int32).reshape(NW, NCHUNK, CHUNK)
    uT3 = user_embedding.T.reshape(NB, 8, V)
    iT3 = item_embedding.T.reshape(NB, 8, V)
    xu = jnp.pad(user_embedding[TPB * 128:].T.reshape(NB, 8, V - TPB * 128),
                 ((0, 0), (0, 0), (0, 128 - (V - TPB * 128))))
    xi = jnp.pad(item_embedding[TPB * 128:].T.reshape(NB, 8, V - TPB * 128),
                 ((0, 0), (0, 0), (0, 128 - (V - TPB * 128))))
    out = _gmf(u, i, uT3, iT3, xu, xi)
    return out.reshape(B, 1)


# speedup vs baseline: 21.2974x; 21.2974x over previous
import jax
import jax.numpy as jnp
from jax import lax
from jax.experimental import pallas as pl
from jax.experimental.pallas import tpu as pltpu
from jax.experimental.pallas import tpu_sc as plsc

NC = 2
NS = 16
NW = NC * NS
L = 16

B = 16384
E = 16
NB = 2
B_PER_W = B // NW
V = 1000000
TPB = V // 128
NT = TPB + 1
FLAT = NB * NT * 1024
CHUNK = 128
NCHUNK = B_PER_W // CHUNK
BATCH = 8


def _detile_body(uT_hbm, iT_hbm, xu_hbm, xi_hbm, uimg_hbm, iimg_hbm,
                 ustage, istage, sem):
    wid = lax.axis_index("s") * NC + lax.axis_index("c")

    def tiles(n):
        out = []
        for t in range(BATCH):
            j = wid + 32 * (BATCH * n + t)
            out.append((t, lax.min(j, TPB - 1)))
        return out

    def step(n, carry):
        copies = []
        for t, jc in tiles(n):
            for b in range(NB):
                copies.append(pltpu.async_copy(
                    uT_hbm.at[b, :, pl.ds(jc * 128, 128)],
                    ustage.at[t, b], sem))
                copies.append(pltpu.async_copy(
                    iT_hbm.at[b, :, pl.ds(jc * 128, 128)],
                    istage.at[t, b], sem))
        for cp in copies:
            cp.wait()
        copies = []
        for t, jc in tiles(n):
            for b in range(NB):
                copies.append(pltpu.async_copy(
                    ustage.at[t, b], uimg_hbm.at[b * NT + jc], sem))
                copies.append(pltpu.async_copy(
                    istage.at[t, b], iimg_hbm.at[b * NT + jc], sem))
        for cp in copies:
            cp.wait()
        return carry

    lax.fori_loop(0, (TPB + 32 * BATCH - 1) // (32 * BATCH), step, 0)

    @pl.when(wid == 0)
    def _():
        for b in range(NB):
            pltpu.async_copy(xu_hbm.at[b], uimg_hbm.at[b * NT + TPB],
                             sem).wait()
            pltpu.async_copy(xi_hbm.at[b], iimg_hbm.at[b * NT + TPB],
                             sem).wait()


def _gather_body(user_hbm, item_hbm, uflat_hbm, iflat_hbm, out_hbm,
                 uvidx, ividx, uaddr, iaddr, ugat, igat, outv, *sems):
    wid = lax.axis_index("s") * NC + lax.axis_index("c")
    base = wid * B_PER_W

    pltpu.sync_copy(user_hbm.at[wid], uvidx)
    pltpu.sync_copy(item_hbm.at[wid], ividx)

    econst = [(e // 8) * (NT * 1024) + (e % 8) * 128 for e in range(E)]

    def build(c, carry):
        for kk in range(CHUNK // L):
            sl = pl.ds(kk * L, L)
            for tab, idxr, addrr in ((0, uvidx, uaddr), (1, ividx, iaddr)):
                rvec = idxr[c, sl]
                t = lax.shift_right_logical(rvec, 7) * 1024 + (rvec & 127)
                for e in range(E):
                    addrr[c, e, sl] = t + econst[e]
        return carry

    lax.fori_loop(0, NCHUNK, build, 0)

    copies = []
    for c in range(NCHUNK):
        cs = []
        for e in range(E):
            cs.append(pltpu.async_copy(
                uflat_hbm.at[uaddr.at[c, e]], ugat.at[c, e], sems[c]))
            cs.append(pltpu.async_copy(
                iflat_hbm.at[iaddr.at[c, e]], igat.at[c, e], sems[c]))
        copies.append(cs)

    def group(c, g, carry):
        sl = pl.ds(g * L, L)
        acc = ugat[c, 0, sl] * igat[c, 0, sl]
        for e in range(1, E):
            acc = acc + ugat[c, e, sl] * igat[c, e, sl]
        outv[pl.ds(c * CHUNK + g * L, L)] = 1.0 / (1.0 + jnp.exp(-acc))
        return carry

    for c in range(NCHUNK):
        for cp in copies[c]:
            cp.wait()
        lax.fori_loop(0, CHUNK // L,
                      lambda g, k, _c=c: group(_c, g, k), 0)

    pltpu.sync_copy(outv, out_hbm.at[pl.ds(base, B_PER_W)])


def _mesh():
    return plsc.VectorSubcoreMesh(
        core_axis_name="c", subcore_axis_name="s",
        num_cores=NC, num_subcores=NS)


@jax.jit
def _gmf(user, item, uT3, iT3, xu, xi):
    uimg, iimg = pl.kernel(
        _detile_body,
        out_type=(jax.ShapeDtypeStruct((NB * NT, 8, 128), jnp.float32),
                  jax.ShapeDtypeStruct((NB * NT, 8, 128), jnp.float32)),
        mesh=_mesh(),
        scratch_types=[
            pltpu.VMEM((BATCH, NB, 8, 128), jnp.float32),
            pltpu.VMEM((BATCH, NB, 8, 128), jnp.float32),
            pltpu.SemaphoreType.DMA,
        ],
        compiler_params=pltpu.CompilerParams(use_tc_tiling_on_sc=True),
    )(uT3, iT3, xu, xi)

    out = pl.kernel(
        _gather_body,
        out_type=jax.ShapeDtypeStruct((B,), jnp.float32),
        mesh=_mesh(),
        scratch_types=[
            pltpu.VMEM((NCHUNK, CHUNK), jnp.int32),
            pltpu.VMEM((NCHUNK, CHUNK), jnp.int32),
            pltpu.VMEM((NCHUNK, E, CHUNK), jnp.int32),
            pltpu.VMEM((NCHUNK, E, CHUNK), jnp.int32),
            pltpu.VMEM((NCHUNK, E, CHUNK), jnp.float32),
            pltpu.VMEM((NCHUNK, E, CHUNK), jnp.float32),
            pltpu.VMEM((B_PER_W,), jnp.float32),
        ] + [pltpu.SemaphoreType.DMA] * NCHUNK,
        compiler_params=pltpu.CompilerParams(use_tc_tiling_on_sc=False),
    )(user, item, uimg.reshape(FLAT), iimg.reshape(FLAT))
    return out


def kernel(user, item, user_embedding, item_embedding):
    u = user.astype(jnp.int32).reshape(NW, NCHUNK, CHUNK)
    i = item.astype(jnp.int32).reshape(NW, NCHUNK, CHUNK)
    uT3 = user_embedding.T.reshape(NB, 8, V)
    iT3 = item_embedding.T.reshape(NB, 8, V)
    xu = jnp.pad(user_embedding[TPB * 128:].T.reshape(NB, 8, V - TPB * 128),
                 ((0, 0), (0, 0), (0, 128 - (V - TPB * 128))))
    xi = jnp.pad(item_embedding[TPB * 128:].T.reshape(NB, 8, V - TPB * 128),
                 ((0, 0), (0, 0), (0, 128 - (V - TPB * 128))))
    out = _gmf(u, i, uT3, iT3, xu, xi)
    return out.reshape(B, 1)
